# baseline (speedup 1.0000x reference)
import functools

import jax
import jax.numpy as jnp
from jax import lax
from jax.experimental import pallas as pl
from jax.experimental.pallas import tpu as pltpu
from jax.experimental.pallas import tpu_sc as plsc

_VOCAB = 8192
_BATCH = 4096
_D = 8192

_info = plsc.get_sparse_core_info()
_NC = _info.num_cores
_NS = _info.num_subcores
_NW = _NC * _NS
_BPW = _BATCH // _NW
_L = _info.num_lanes
_G = _BPW // _L

_mesh = plsc.VectorSubcoreMesh(core_axis_name="c", subcore_axis_name="s")


@functools.partial(
    pl.kernel,
    mesh=_mesh,
    out_type=jax.ShapeDtypeStruct((_BATCH, _D), jnp.float32),
    scratch_types=[
        pltpu.VMEM((_BPW,), jnp.int32),
        pltpu.SemaphoreType.DMA,
    ],
)
def _sc_gather(idx_hbm, table_hbm, out_hbm, idx_v, sem):
    wid = lax.axis_index("s") * _NC + lax.axis_index("c")
    base = wid * _BPW
    pltpu.sync_copy(idx_hbm.at[wid], idx_v)
    lanes = lax.iota(jnp.int32, _L)

    def group(g, carry):
        vec = idx_v[pl.ds(g * _L, _L)]
        for j in range(_L):
            s = vec[j]
            pltpu.async_copy(
                table_hbm.at[pl.ds(s, 1)],
                out_hbm.at[pl.ds(base + g * _L + j, 1)],
                sem,
            )
        @pl.when(g >= 1)
        def _drain():
            for j in range(_L):
                pltpu.make_async_copy(
                    table_hbm.at[pl.ds(0, 1)], out_hbm.at[pl.ds(base, 1)], sem
                ).wait()

        return carry

    lax.fori_loop(0, _G, group, 0)
    for j in range(_L):
        pltpu.make_async_copy(
            table_hbm.at[pl.ds(0, 1)], out_hbm.at[pl.ds(base, 1)], sem
        ).wait()


def kernel(x, table):
    idx = x.astype(jnp.int32).reshape(_NW, _BPW)
    return _sc_gather(idx, table)


# speedup vs baseline: 36.4050x; 36.4050x over previous
import functools

import jax
import jax.numpy as jnp
from jax import lax
from jax.experimental import pallas as pl
from jax.experimental.pallas import tpu as pltpu
from jax.experimental.pallas import tpu_sc as plsc

_VOCAB = 8192
_BATCH = 4096
_D = 8192

_info = plsc.get_sparse_core_info()
_NC = _info.num_cores
_NS = _info.num_subcores
_NW = _NC * _NS
_BPW = _BATCH // _NW
_R = 4
_CH = _BPW // _R
_NB = 2

_mesh = plsc.VectorSubcoreMesh(core_axis_name="c", subcore_axis_name="s")


@functools.partial(
    pl.kernel,
    mesh=_mesh,
    out_type=jax.ShapeDtypeStruct((_BATCH, _D), jnp.float32),
    scratch_types=[
        pltpu.VMEM((_CH, _R), jnp.int32),
        pltpu.VMEM((_R, _D), jnp.float32),
        pltpu.VMEM((_R, _D), jnp.float32),
        pltpu.SemaphoreType.DMA,
        pltpu.SemaphoreType.DMA,
        pltpu.SemaphoreType.DMA,
        pltpu.SemaphoreType.DMA,
    ],
)
def _sc_gather(idx_hbm, table_hbm, out_hbm, idx_v, buf0, buf1, g0, g1, w0, w1):
    wid = lax.axis_index("s") * _NC + lax.axis_index("c")
    base = wid * _BPW
    pltpu.sync_copy(idx_hbm.at[wid], idx_v)
    bufs = (buf0, buf1)
    gsems = (g0, g1)
    wsems = (w0, w1)

    for b in range(_NB):
        pltpu.async_copy(table_hbm.at[idx_v.at[b]], bufs[b], gsems[b])

    def step(g, carry):
        for b in range(_NB):
            c = g * _NB + b
            pltpu.make_async_copy(table_hbm.at[idx_v.at[c]], bufs[b], gsems[b]).wait()
            pltpu.async_copy(bufs[b], out_hbm.at[pl.ds(base + c * _R, _R)], wsems[b])
            pltpu.make_async_copy(bufs[b], out_hbm.at[pl.ds(base + c * _R, _R)], wsems[b]).wait()
            nxt = c + _NB

            @pl.when(nxt < _CH)
            def _start_next():
                pltpu.async_copy(table_hbm.at[idx_v.at[nxt]], bufs[b], gsems[b])

        return carry

    lax.fori_loop(0, _CH // _NB, step, 0)


def kernel(x, table):
    idx = x.astype(jnp.int32).reshape(_NW, _CH, _R)
    return _sc_gather(idx, table)


# speedup vs baseline: 36.6926x; 1.0079x over previous
import functools

import jax
import jax.numpy as jnp
from jax import lax
from jax.experimental import pallas as pl
from jax.experimental.pallas import tpu as pltpu
from jax.experimental.pallas import tpu_sc as plsc

_VOCAB = 8192
_BATCH = 4096
_D = 8192

_info = plsc.get_sparse_core_info()
_NC = _info.num_cores
_NS = _info.num_subcores
_NW = _NC * _NS
_BPW = _BATCH // _NW
_R = 4
_CH = _BPW // _R
_NB = 2

_mesh = plsc.VectorSubcoreMesh(core_axis_name="c", subcore_axis_name="s")


@functools.partial(
    pl.kernel,
    mesh=_mesh,
    out_type=jax.ShapeDtypeStruct((_BATCH, _D), jnp.float32),
    scratch_types=[
        pltpu.VMEM((_CH, _R), jnp.int32),
        pltpu.VMEM((_R, _D), jnp.float32),
        pltpu.VMEM((_R, _D), jnp.float32),
        pltpu.SemaphoreType.DMA,
        pltpu.SemaphoreType.DMA,
        pltpu.SemaphoreType.DMA,
        pltpu.SemaphoreType.DMA,
    ],
)
def _sc_gather(idx_hbm, table_hbm, out_hbm, idx_v, buf0, buf1, g0, g1, w0, w1):
    wid = lax.axis_index("s") * _NC + lax.axis_index("c")
    pltpu.sync_copy(idx_hbm.at[wid], idx_v)
    bufs = (buf0, buf1)
    gsems = (g0, g1)
    wsems = (w0, w1)

    for b in range(_NB):
        pltpu.async_copy(table_hbm.at[idx_v.at[b]], bufs[b], gsems[b])

    def step(g, carry):
        for b in range(_NB):
            c = g * _NB + b
            pltpu.make_async_copy(table_hbm.at[idx_v.at[c]], bufs[b], gsems[b]).wait()
            obase = (c * _NW + wid) * _R
            pltpu.async_copy(bufs[b], out_hbm.at[pl.ds(obase, _R)], wsems[b])
            pltpu.make_async_copy(bufs[b], out_hbm.at[pl.ds(obase, _R)], wsems[b]).wait()
            nxt = c + _NB

            @pl.when(nxt < _CH)
            def _start_next():
                pltpu.async_copy(table_hbm.at[idx_v.at[nxt]], bufs[b], gsems[b])

        return carry

    lax.fori_loop(0, _CH // _NB, step, 0)


def kernel(x, table):
    idx = x.astype(jnp.int32).reshape(_CH, _NW, _R).transpose(1, 0, 2)
    return _sc_gather(idx, table)
